# baseline (speedup 1.0000x reference)
import functools

import jax
import jax.numpy as jnp
from jax import lax
from jax.experimental import pallas as pl
from jax.experimental.pallas import tpu as pltpu

_N = 1024
_E = 16384
_ET = _E + _N
_BE = 512
_NB = _ET // _BE
_BI = 8
_NBI = _N // _BI
_EPS = 1e-5


def _lrelu(v, slope):
    return jnp.where(v >= 0, v, slope * v)


def _gat_layer(x, s_cols, d_cols, d_rows, ea_cols, ear, Wl, Wr, Att, Exp, We,
               bias, g, be, do_elu):
    F = x.shape[1]
    C = Wl.shape[1]
    H = Att.shape[1]

    def body(s_ref, dc_ref, dr_ref, ea_ref, ear_ref, x_ref, wl_ref, wr_ref,
             att_ref, exp_ref, we_ref, b_ref, g_ref, be_ref, out_ref,
             xl_s, xr_s, num_s, den_s, mean_s):
        k = pl.program_id(0)

        @pl.when(k == 0)
        def _():
            xv = x_ref[...]
            xl_s[...] = jnp.dot(xv, wl_ref[...],
                                preferred_element_type=jnp.float32)
            xr_s[...] = jnp.dot(xv, wr_ref[...],
                                preferred_element_type=jnp.float32)
            num_s[...] = jnp.zeros_like(num_s)
            den_s[...] = jnp.zeros_like(den_s)
            mean_s[0, 0] = jnp.sum(ear_ref[...]) / _E

        s = s_ref[0]
        dcol = dc_ref[0]
        drow = dr_ref[0]
        eid = k * _BE + lax.broadcasted_iota(jnp.int32, (_BE, 1), 0)
        ea = jnp.where(eid < _E, ea_ref[0], mean_s[0, 0])

        iota_en = lax.broadcasted_iota(jnp.int32, (_BE, _N), 1)
        iota_ne = lax.broadcasted_iota(jnp.int32, (_N, _BE), 0)
        Gs = (s == iota_en).astype(jnp.float32)
        Gd = (dcol == iota_en).astype(jnp.float32)
        Gdt = (drow == iota_ne).astype(jnp.float32)

        xle = jnp.dot(Gs, xl_s[...], preferred_element_type=jnp.float32)
        xre = jnp.dot(Gd, xr_s[...], preferred_element_type=jnp.float32)
        m = _lrelu(xle + xre + ea * we_ref[...], 0.2)
        logits = jnp.dot(m, att_ref[...], preferred_element_type=jnp.float32)
        exl = jnp.exp(logits)
        exl_c = jnp.dot(exl, exp_ref[...], preferred_element_type=jnp.float32)
        num_s[...] += jnp.dot(Gdt, xle * exl_c,
                              preferred_element_type=jnp.float32)
        den_s[...] += jnp.dot(Gdt, exl, preferred_element_type=jnp.float32)

        @pl.when(k == _NB - 1)
        def _():
            den_c = jnp.dot(den_s[...], exp_ref[...],
                            preferred_element_type=jnp.float32)
            o = num_s[...] / den_c + b_ref[...]
            mu = jnp.mean(o, axis=-1, keepdims=True)
            var = jnp.mean((o - mu) * (o - mu), axis=-1, keepdims=True)
            o = (o - mu) * lax.rsqrt(var + _EPS) * g_ref[...] + be_ref[...]
            if do_elu:
                o = jnp.where(o > 0, o, jnp.exp(o) - 1.0)
            out_ref[...] = o

    full = lambda shape: pl.BlockSpec(shape, lambda k: tuple(0 for _ in shape))
    return pl.pallas_call(
        body,
        grid=(_NB,),
        in_specs=[
            pl.BlockSpec((1, _BE, 1), lambda k: (k, 0, 0)),
            pl.BlockSpec((1, _BE, 1), lambda k: (k, 0, 0)),
            pl.BlockSpec((1, 1, _BE), lambda k: (k, 0, 0)),
            pl.BlockSpec((1, _BE, 1), lambda k: (k, 0, 0)),
            full((1, _E)),
            full((_N, F)),
            full((F, C)),
            full((F, C)),
            full((C, H)),
            full((H, C)),
            full((1, C)),
            full((1, C)),
            full((1, C)),
            full((1, C)),
        ],
        out_specs=full((_N, C)),
        out_shape=jax.ShapeDtypeStruct((_N, C), jnp.float32),
        scratch_shapes=[
            pltpu.VMEM((_N, C), jnp.float32),
            pltpu.VMEM((_N, C), jnp.float32),
            pltpu.VMEM((_N, C), jnp.float32),
            pltpu.VMEM((_N, H), jnp.float32),
            pltpu.SMEM((1, 1), jnp.float32),
        ],
        compiler_params=pltpu.CompilerParams(
            dimension_semantics=("arbitrary",)),
    )(s_cols, d_cols, d_rows, ea_cols, ear, x, Wl, Wr, Att, Exp, We,
      bias, g, be)


def _decode(emb, embT, D1a, D1aT, D1bT, d1b_c, d1b_r, ga_c, ba_c, D2T, d2b_c,
            gb_c, bb_c, D3_r, d3b_c):
    def body(emb_ref, embT_ref, d1a_ref, d1aT_ref, d1bT_ref, d1bias_ref,
             d1bias_r_ref, ga_ref, ba_ref, d2_ref, d2b_ref, gb_ref, bb_ref,
             d3_ref, d3b_ref, out_ref, Ac_s, Ar_s, Bt_s, Btg_s, varB_s):
        k = pl.program_id(0)

        @pl.when(k == 0)
        def _():
            et = embT_ref[...]
            Ac_s[...] = jnp.dot(d1aT_ref[...], et,
                                preferred_element_type=jnp.float32) \
                + d1bias_ref[...]
            Ar_s[...] = jnp.dot(emb_ref[...], d1a_ref[...],
                                preferred_element_type=jnp.float32) \
                + d1bias_r_ref[...]
            B = jnp.dot(d1bT_ref[...], et,
                        preferred_element_type=jnp.float32)
            muB = jnp.mean(B, axis=0, keepdims=True)
            Bt = B - muB
            Bt_s[...] = Bt
            Btg_s[...] = Bt * ga_ref[...]
            varB_s[...] = jnp.mean(Bt * Bt, axis=0, keepdims=True)

        Bt = Bt_s[...]
        Btg = Btg_s[...]
        varB = varB_s[...]
        ga_v = ga_ref[...]
        ba_v = ba_ref[...]
        gb_v = gb_ref[...]
        bb_v = bb_ref[...]
        d2b_v = d2b_ref[...]
        inv32 = 1.0 / 32.0
        iota_col = lax.broadcasted_iota(jnp.int32, (_N, 1), 0)
        iota_row = lax.broadcasted_iota(jnp.int32, (1, _N), 1)
        for r in range(_BI):
            i = k * _BI + r
            oh_c = (iota_col == i).astype(jnp.float32)
            oh_r = (iota_row == i).astype(jnp.float32)
            a_c = jnp.dot(Ac_s[...], oh_c,
                          preferred_element_type=jnp.float32)
            a_r = jnp.dot(oh_r, Ar_s[...],
                          preferred_element_type=jnp.float32)
            muA = jnp.mean(a_r, axis=1, keepdims=True)
            at_c = a_c - muA
            at_r = a_r - muA
            vA = jnp.mean(at_r * at_r, axis=1, keepdims=True)
            cross = jnp.dot(at_r, Bt,
                            preferred_element_type=jnp.float32) * (2.0 / 64.0)
            scale = lax.rsqrt(varB + cross + vA + _EPS)
            h = (Btg + at_c * ga_v) * scale + ba_v
            h = jnp.maximum(h, 0.1 * h)
            h2 = jnp.dot(d2_ref[...], h,
                         preferred_element_type=jnp.float32) + d2b_v
            mu2 = jnp.mean(h2, axis=0, keepdims=True)
            c2 = h2 - mu2
            var2 = jnp.mean(c2 * c2, axis=0, keepdims=True)
            h2 = c2 * lax.rsqrt(var2 + _EPS) * gb_v + bb_v
            h2 = jnp.maximum(h2, 0.1 * h2)
            logit = jnp.dot(d3_ref[...], h2,
                            preferred_element_type=jnp.float32) + d3b_ref[...]
            out_ref[r:r + 1, :] = 1.0 / (1.0 + jnp.exp(-logit))

    full = lambda shape: pl.BlockSpec(shape, lambda k: tuple(0 for _ in shape))
    return pl.pallas_call(
        body,
        grid=(_NBI,),
        in_specs=[
            full((_N, 32)),
            full((32, _N)),
            full((32, 64)),
            full((64, 32)),
            full((64, 32)),
            full((64, 1)),
            full((1, 64)),
            full((64, 1)),
            full((64, 1)),
            full((32, 64)),
            full((32, 1)),
            full((32, 1)),
            full((32, 1)),
            full((1, 32)),
            full((1, 1)),
        ],
        out_specs=pl.BlockSpec((_BI, _N), lambda k: (k, 0)),
        out_shape=jax.ShapeDtypeStruct((_N, _N), jnp.float32),
        scratch_shapes=[
            pltpu.VMEM((64, _N), jnp.float32),
            pltpu.VMEM((_N, 64), jnp.float32),
            pltpu.VMEM((64, _N), jnp.float32),
            pltpu.VMEM((64, _N), jnp.float32),
            pltpu.VMEM((1, _N), jnp.float32),
        ],
        compiler_params=pltpu.CompilerParams(
            dimension_semantics=("arbitrary",)),
    )(emb, embT, D1a, D1aT, D1bT, d1b_c, d1b_r, ga_c, ba_c, D2T, d2b_c,
      gb_c, bb_c, D3_r, d3b_c)


def _att_mats(a, heads, ch):
    C = heads * ch
    mask = (jnp.arange(C)[:, None] // ch) == jnp.arange(heads)[None, :]
    Att = jnp.where(mask, a.reshape(C, 1), 0.0).astype(jnp.float32)
    Exp = mask.T.astype(jnp.float32)
    return Att, Exp


@jax.jit
def kernel(x, edge_index, edge_attr, W1l, W1r, a1, We1, b1, W2l, W2r, a2, We2,
           b2, W3l, W3r, a3, We3, b3, g1, be1, g2, be2, g3, be3, D1, d1b, ga,
           ba, D2, d2b, gb, bb, D3, d3b):
    src, dst = edge_index[0], edge_index[1]
    loop = jnp.arange(_N, dtype=src.dtype)
    s2 = jnp.concatenate([src, loop])
    d2 = jnp.concatenate([dst, loop])
    ea2 = jnp.concatenate([edge_attr[:, 0], jnp.zeros(_N, jnp.float32)])

    s_cols = s2.reshape(_NB, _BE, 1)
    d_cols = d2.reshape(_NB, _BE, 1)
    d_rows = d2.reshape(_NB, 1, _BE)
    ea_cols = ea2.reshape(_NB, _BE, 1)
    ear = edge_attr.reshape(1, _E)

    row = lambda v: v.reshape(1, -1)
    Att1, Exp1 = _att_mats(a1, 4, 16)
    Att2, Exp2 = _att_mats(a2, 4, 16)
    Att3, Exp3 = _att_mats(a3, 1, 32)

    h = _gat_layer(x, s_cols, d_cols, d_rows, ea_cols, ear, W1l, W1r,
                   Att1, Exp1, We1, row(b1), row(g1), row(be1), True)
    h = _gat_layer(h, s_cols, d_cols, d_rows, ea_cols, ear, W2l, W2r,
                   Att2, Exp2, We2, row(b2), row(g2), row(be2), True)
    emb = _gat_layer(h, s_cols, d_cols, d_rows, ea_cols, ear, W3l, W3r,
                     Att3, Exp3, We3, row(b3), row(g3), row(be3), False)

    col = lambda v: v.reshape(-1, 1)
    weights = _decode(emb, emb.T, D1[:32], D1[:32].T, D1[32:].T, col(d1b),
                      row(d1b), col(ga), col(ba), D2.T, col(d2b), col(gb),
                      col(bb), D3.reshape(1, 32), d3b.reshape(1, 1))
    return (weights, emb)


# speedup vs baseline: 1.4613x; 1.4613x over previous
import functools

import jax
import jax.numpy as jnp
from jax import lax
from jax.experimental import pallas as pl
from jax.experimental.pallas import tpu as pltpu
from jax.experimental.pallas import tpu_sc as plsc

_N = 1024
_E = 16384
_ET = _E + _N
_NW = 32
_NCH = (_ET // _NW) // 16
_BI = 8
_NBI = _N // _BI
_EPS = 1e-5


def _sc_edge_pass(xlr, s3, d3, ea3, attB, WeB, heads, C):
    H = heads
    ch = C // H
    mesh = plsc.VectorSubcoreMesh(core_axis_name="c", subcore_axis_name="s")

    @functools.partial(
        pl.kernel,
        mesh=mesh,
        out_type=jax.ShapeDtypeStruct((2, _N, 128), jnp.float32),
        scratch_types=[
            pltpu.VMEM((_NCH, 16), jnp.int32),
            pltpu.VMEM((_NCH, 16), jnp.int32),
            pltpu.VMEM((_NCH, 16), jnp.float32),
            pltpu.VMEM((C, 16), jnp.float32),
            pltpu.VMEM((C, 16), jnp.float32),
            pltpu.VMEM((16, 128), jnp.float32),
            pltpu.VMEM((16, 128), jnp.float32),
            pltpu.VMEM((16, 128), jnp.float32),
            pltpu.VMEM_SHARED((_N, 128), jnp.float32),
            pltpu.SemaphoreType.DMA,
            pltpu.SemaphoreType.DMA,
        ],
        compiler_params=pltpu.CompilerParams(needs_layout_passes=False),
    )
    def k(xlr_hbm, s_hbm, d_hbm, ea_hbm, attB_hbm, WeB_hbm, acc_hbm,
          s_v, d_v, ea_v, attB_v, WeB_v, xs_b, xd_b, n_b, acc_sh,
          sem1, sem2):
        cid = lax.axis_index("c")
        sid = lax.axis_index("s")
        wid = sid * 2 + cid

        pltpu.sync_copy(s_hbm.at[wid], s_v)
        pltpu.sync_copy(d_hbm.at[wid], d_v)
        pltpu.sync_copy(ea_hbm.at[wid], ea_v)
        pltpu.sync_copy(attB_hbm, attB_v)
        pltpu.sync_copy(WeB_hbm, WeB_v)

        zero = jnp.zeros((16,), jnp.float32)
        for r in range(16):
            for cc in range(8):
                n_b[r, pl.ds(cc * 16, 16)] = zero
        for q in range(4):
            pltpu.sync_copy(n_b, acc_sh.at[pl.ds(sid * 64 + q * 16, 16)])
        plsc.subcore_barrier()

        lane = lax.iota(jnp.int32, 16)

        def chunk(j, carry):
            sv = s_v[j]
            dv = d_v[j]
            eav = ea_v[j]
            cp1 = pltpu.async_copy(xlr_hbm.at[sv], xs_b, sem1)
            cp2 = pltpu.async_copy(xlr_hbm.at[dv], xd_b, sem2)
            cp1.wait()
            cp2.wait()
            logits = [jnp.zeros((16,), jnp.float32) for _ in range(H)]
            for c in range(C):
                cl = jnp.full((16,), c, jnp.int32)
                cr = jnp.full((16,), 64 + c, jnp.int32)
                xlc = plsc.load_gather(xs_b, [lane, cl])
                xrc = plsc.load_gather(xd_b, [lane, cr])
                mc = xlc + xrc + eav * WeB_v[c]
                mc = jnp.maximum(mc, 0.2 * mc)
                logits[c // ch] = logits[c // ch] + mc * attB_v[c]
            exl = [jnp.exp(lg) for lg in logits]
            for c in range(C):
                cl = jnp.full((16,), c, jnp.int32)
                xlc = plsc.load_gather(xs_b, [lane, cl])
                plsc.store_scatter(n_b, [lane, cl], xlc * exl[c // ch])
            for h in range(H):
                ch_ = jnp.full((16,), 64 + h, jnp.int32)
                plsc.store_scatter(n_b, [lane, ch_], exl[h])
            pltpu.sync_copy(n_b, acc_sh.at[d_v.at[j]], add=True)
            return carry

        lax.fori_loop(0, _NCH, chunk, 0)
        plsc.subcore_barrier()
        rows = pl.ds(sid * 64, 64)
        pltpu.sync_copy(acc_sh.at[rows], acc_hbm.at[cid, rows])

    return k(xlr, s3, d3, ea3, attB, WeB)


def _tc_full(body, in_shapes, out_shape, n_out=1):
    full = lambda shape: pl.BlockSpec(shape, lambda: tuple(0 for _ in shape))
    outs = out_shape if isinstance(out_shape, tuple) else (out_shape,)
    return pl.pallas_call(
        body,
        in_specs=[full(s) for s in in_shapes],
        out_specs=[full(s.shape) for s in outs] if n_out > 1
        else full(outs[0].shape),
        out_shape=out_shape,
    )


def _proj1(x, Wl, Wr, ear):
    def body(x_ref, wl_ref, wr_ref, ear_ref, xlr_ref, mean_ref):
        xv = x_ref[...]
        xl = jnp.dot(xv, wl_ref[...], preferred_element_type=jnp.float32)
        xr = jnp.dot(xv, wr_ref[...], preferred_element_type=jnp.float32)
        xlr_ref[...] = jnp.concatenate([xl, xr], axis=1)
        mean_ref[...] = jnp.sum(ear_ref[...], axis=1,
                                keepdims=True) * (1.0 / _E)

    return _tc_full(
        body,
        [x.shape, Wl.shape, Wr.shape, ear.shape],
        (jax.ShapeDtypeStruct((_N, 128), jnp.float32),
         jax.ShapeDtypeStruct((1, 1), jnp.float32)),
        n_out=2,
    )(x, Wl, Wr, ear)


def _node_proj(acc, Snum, Sden, b, g, be, Wl, Wr, pad):
    C2 = Wl.shape[1]

    def body(acc_ref, sn_ref, sd_ref, b_ref, g_ref, be_ref, wl_ref, wr_ref,
             xlr_ref):
        accs = acc_ref[0] + acc_ref[1]
        num = jnp.dot(accs, sn_ref[...], preferred_element_type=jnp.float32)
        den = jnp.dot(accs, sd_ref[...], preferred_element_type=jnp.float32)
        o = num / den + b_ref[...]
        mu = jnp.mean(o, axis=-1, keepdims=True)
        var = jnp.mean((o - mu) * (o - mu), axis=-1, keepdims=True)
        o = (o - mu) * lax.rsqrt(var + _EPS) * g_ref[...] + be_ref[...]
        o = jnp.where(o > 0, o, jnp.exp(o) - 1.0)
        xl = jnp.dot(o, wl_ref[...], preferred_element_type=jnp.float32)
        xr = jnp.dot(o, wr_ref[...], preferred_element_type=jnp.float32)
        if pad:
            z = jnp.zeros_like(xl)
            xlr_ref[...] = jnp.concatenate([xl, z, xr, z], axis=1)
        else:
            xlr_ref[...] = jnp.concatenate([xl, xr], axis=1)

    return _tc_full(
        body,
        [acc.shape, Snum.shape, Sden.shape, b.shape, g.shape, be.shape,
         Wl.shape, Wr.shape],
        jax.ShapeDtypeStruct((_N, 128), jnp.float32),
    )(acc, Snum, Sden, b, g, be, Wl, Wr)


def _node_final(acc, Snum, Sden, b, g, be):
    def body(acc_ref, sn_ref, sd_ref, b_ref, g_ref, be_ref, emb_ref):
        accs = acc_ref[0] + acc_ref[1]
        num = jnp.dot(accs, sn_ref[...], preferred_element_type=jnp.float32)
        den = jnp.dot(accs, sd_ref[...], preferred_element_type=jnp.float32)
        o = num / den + b_ref[...]
        mu = jnp.mean(o, axis=-1, keepdims=True)
        var = jnp.mean((o - mu) * (o - mu), axis=-1, keepdims=True)
        emb_ref[...] = (o - mu) * lax.rsqrt(var + _EPS) * g_ref[...] \
            + be_ref[...]

    return _tc_full(
        body,
        [acc.shape, Snum.shape, Sden.shape, b.shape, g.shape, be.shape],
        jax.ShapeDtypeStruct((_N, 32), jnp.float32),
    )(acc, Snum, Sden, b, g, be)


def _decode(embT, D1aT, D1bT, d1b_c, ga_c, ba_c, D2T, d2b_c, gb_c, bb_c,
            D3_c, d3b_c):
    def body(embT_ref, d1a_ref, d1b_ref, d1bias_ref, ga_ref, ba_ref, d2_ref,
             d2b_ref, gb_ref, bb_ref, d3_ref, d3b_ref, out_ref, A_s, B_s):
        k = pl.program_id(0)

        @pl.when(k == 0)
        def _():
            et = embT_ref[...]
            A_s[...] = jnp.dot(d1a_ref[...], et,
                               preferred_element_type=jnp.float32) \
                + d1bias_ref[...]
            B_s[...] = jnp.dot(d1b_ref[...], et,
                               preferred_element_type=jnp.float32)

        B = B_s[...]
        Afull = A_s[...]
        ga_v = ga_ref[...]
        ba_v = ba_ref[...]
        gb_v = gb_ref[...]
        bb_v = bb_ref[...]
        d2b_v = d2b_ref[...]
        d3_v = d3_ref[...]
        iota_cols = lax.broadcasted_iota(jnp.int32, (64, _N), 1)
        for r in range(_BI):
            sel = iota_cols == (k * _BI + r)
            a = jnp.sum(jnp.where(sel, Afull, 0.0), axis=1, keepdims=True)
            h = B + a
            mu = jnp.mean(h, axis=0, keepdims=True)
            var = jnp.mean((h - mu) * (h - mu), axis=0, keepdims=True)
            h = (h - mu) * lax.rsqrt(var + _EPS) * ga_v + ba_v
            h = jnp.maximum(h, 0.1 * h)
            h2 = jnp.dot(d2_ref[...], h,
                         preferred_element_type=jnp.float32) + d2b_v
            mu2 = jnp.mean(h2, axis=0, keepdims=True)
            var2 = jnp.mean((h2 - mu2) * (h2 - mu2), axis=0, keepdims=True)
            h2 = (h2 - mu2) * lax.rsqrt(var2 + _EPS) * gb_v + bb_v
            h2 = jnp.maximum(h2, 0.1 * h2)
            logit = jnp.sum(h2 * d3_v, axis=0, keepdims=True) + d3b_ref[...]
            out_ref[r:r + 1, :] = 1.0 / (1.0 + jnp.exp(-logit))

    full = lambda shape: pl.BlockSpec(shape, lambda k: tuple(0 for _ in shape))
    return pl.pallas_call(
        body,
        grid=(_NBI,),
        in_specs=[
            full((32, _N)),
            full((64, 32)),
            full((64, 32)),
            full((64, 1)),
            full((64, 1)),
            full((64, 1)),
            full((32, 64)),
            full((32, 1)),
            full((32, 1)),
            full((32, 1)),
            full((32, 1)),
            full((1, 1)),
        ],
        out_specs=pl.BlockSpec((_BI, _N), lambda k: (k, 0)),
        out_shape=jax.ShapeDtypeStruct((_N, _N), jnp.float32),
        scratch_shapes=[
            pltpu.VMEM((64, _N), jnp.float32),
            pltpu.VMEM((64, _N), jnp.float32),
        ],
        compiler_params=pltpu.CompilerParams(
            dimension_semantics=("arbitrary",)),
    )(embT, D1aT, D1bT, d1b_c, ga_c, ba_c, D2T, d2b_c, gb_c, bb_c,
      D3_c, d3b_c)


def _bcast16(v):
    return jnp.broadcast_to(v.reshape(-1, 1), (v.size, 16)) \
        .astype(jnp.float32)


def _sel_mats(heads, C):
    ch = C // heads
    cols = jnp.arange(C)
    rows = jnp.arange(128)
    Snum = (rows[:, None] == cols[None, :]).astype(jnp.float32)
    Sden = ((rows[:, None] - 64) == (cols[None, :] // ch)).astype(jnp.float32)
    return Snum, Sden


@jax.jit
def kernel(x, edge_index, edge_attr, W1l, W1r, a1, We1, b1, W2l, W2r, a2, We2,
           b2, W3l, W3r, a3, We3, b3, g1, be1, g2, be2, g3, be3, D1, d1b, ga,
           ba, D2, d2b, gb, bb, D3, d3b):
    src, dst = edge_index[0], edge_index[1]
    loop = jnp.arange(_N, dtype=src.dtype)
    s3 = jnp.concatenate([src, loop]).reshape(_NW, _NCH, 16)
    d3 = jnp.concatenate([dst, loop]).reshape(_NW, _NCH, 16)
    ear = edge_attr.reshape(1, _E)

    row = lambda v: v.reshape(1, -1)
    col = lambda v: v.reshape(-1, 1)

    xlr1, mean_ea = _proj1(x, W1l, W1r, ear)
    ea3 = jnp.concatenate(
        [edge_attr[:, 0], jnp.broadcast_to(mean_ea.reshape(1), (_N,))]
    ).reshape(_NW, _NCH, 16)

    Snum4, Sden4 = _sel_mats(4, 64)
    Snum1, Sden1 = _sel_mats(1, 32)

    acc1 = _sc_edge_pass(xlr1, s3, d3, ea3, _bcast16(a1), _bcast16(We1),
                         4, 64)
    xlr2 = _node_proj(acc1, Snum4, Sden4, row(b1), row(g1), row(be1),
                      W2l, W2r, False)
    acc2 = _sc_edge_pass(xlr2, s3, d3, ea3, _bcast16(a2), _bcast16(We2),
                         4, 64)
    xlr3 = _node_proj(acc2, Snum4, Sden4, row(b2), row(g2), row(be2),
                      W3l, W3r, True)
    acc3 = _sc_edge_pass(xlr3, s3, d3, ea3, _bcast16(a3), _bcast16(We3),
                         1, 32)
    emb = _node_final(acc3, Snum1, Sden1, row(b3), row(g3), row(be3))

    weights = _decode(emb.T, D1[:32].T, D1[32:].T, col(d1b), col(ga),
                      col(ba), D2.T, col(d2b), col(gb), col(bb),
                      D3.reshape(32, 1), d3b.reshape(1, 1))
    return (weights, emb)


# speedup vs baseline: 1.5760x; 1.0785x over previous
import functools

import jax
import jax.numpy as jnp
from jax import lax
from jax.experimental import pallas as pl
from jax.experimental.pallas import tpu as pltpu
from jax.experimental.pallas import tpu_sc as plsc

_N = 1024
_E = 16384
_ET = _E + _N
_NW = 32
_NCH = (_ET // _NW) // 16
_BI = 8
_NBI = _N // _BI
_EPS = 1e-5


def _sc_edge_pass(xlr, s3, d3, ea3, attB, WeB, heads, C):
    H = heads
    ch = C // H
    mesh = plsc.VectorSubcoreMesh(core_axis_name="c", subcore_axis_name="s")

    @functools.partial(
        pl.kernel,
        mesh=mesh,
        out_type=jax.ShapeDtypeStruct((2, _N, 128), jnp.float32),
        scratch_types=[
            pltpu.VMEM((_NCH, 16), jnp.int32),
            pltpu.VMEM((_NCH, 16), jnp.int32),
            pltpu.VMEM((_NCH, 16), jnp.float32),
            pltpu.VMEM((C, 16), jnp.float32),
            pltpu.VMEM((C, 16), jnp.float32),
            pltpu.VMEM((16, 128), jnp.float32),
            pltpu.VMEM((16, 128), jnp.float32),
            pltpu.VMEM((16, 128), jnp.float32),
            pltpu.VMEM_SHARED((_N, 128), jnp.float32),
            pltpu.VMEM_SHARED((_N, 128), jnp.float32),
            pltpu.SemaphoreType.DMA,
            pltpu.SemaphoreType.DMA,
        ],
        compiler_params=pltpu.CompilerParams(needs_layout_passes=False),
    )
    def k(xlr_hbm, s_hbm, d_hbm, ea_hbm, attB_hbm, WeB_hbm, acc_hbm,
          s_v, d_v, ea_v, attB_v, WeB_v, xs_b, xd_b, n_b, acc_sh, tab_sh,
          sem1, sem2):
        cid = lax.axis_index("c")
        sid = lax.axis_index("s")
        wid = sid * 2 + cid

        pltpu.sync_copy(s_hbm.at[wid], s_v)
        pltpu.sync_copy(d_hbm.at[wid], d_v)
        pltpu.sync_copy(ea_hbm.at[wid], ea_v)
        pltpu.sync_copy(attB_hbm, attB_v)
        pltpu.sync_copy(WeB_hbm, WeB_v)
        my_rows = pl.ds(sid * 64, 64)
        pltpu.sync_copy(xlr_hbm.at[my_rows], tab_sh.at[my_rows])

        zero = jnp.zeros((16,), jnp.float32)
        for r in range(16):
            for cc in range(8):
                n_b[r, pl.ds(cc * 16, 16)] = zero
        for q in range(4):
            pltpu.sync_copy(n_b, acc_sh.at[pl.ds(sid * 64 + q * 16, 16)])
        plsc.subcore_barrier()

        lane = lax.iota(jnp.int32, 16)

        def chunk(j, carry):
            sv = s_v[j]
            dv = d_v[j]
            eav = ea_v[j]
            cp1 = pltpu.async_copy(tab_sh.at[sv], xs_b, sem1)
            cp2 = pltpu.async_copy(tab_sh.at[dv], xd_b, sem2)
            cp1.wait()
            cp2.wait()
            logits = [jnp.zeros((16,), jnp.float32) for _ in range(H)]
            for c in range(C):
                cl = jnp.full((16,), c, jnp.int32)
                cr = jnp.full((16,), 64 + c, jnp.int32)
                xlc = plsc.load_gather(xs_b, [lane, cl])
                xrc = plsc.load_gather(xd_b, [lane, cr])
                mc = xlc + xrc + eav * WeB_v[c]
                mc = jnp.maximum(mc, 0.2 * mc)
                logits[c // ch] = logits[c // ch] + mc * attB_v[c]
            exl = [jnp.exp(lg) for lg in logits]
            for c in range(C):
                cl = jnp.full((16,), c, jnp.int32)
                xlc = plsc.load_gather(xs_b, [lane, cl])
                plsc.store_scatter(n_b, [lane, cl], xlc * exl[c // ch])
            for h in range(H):
                ch_ = jnp.full((16,), 64 + h, jnp.int32)
                plsc.store_scatter(n_b, [lane, ch_], exl[h])
            pltpu.sync_copy(n_b, acc_sh.at[d_v.at[j]], add=True)
            return carry

        lax.fori_loop(0, _NCH, chunk, 0)
        plsc.subcore_barrier()
        rows = pl.ds(sid * 64, 64)
        pltpu.sync_copy(acc_sh.at[rows], acc_hbm.at[cid, rows])

    return k(xlr, s3, d3, ea3, attB, WeB)


def _tc_full(body, in_shapes, out_shape, n_out=1):
    full = lambda shape: pl.BlockSpec(shape, lambda: tuple(0 for _ in shape))
    outs = out_shape if isinstance(out_shape, tuple) else (out_shape,)
    return pl.pallas_call(
        body,
        in_specs=[full(s) for s in in_shapes],
        out_specs=[full(s.shape) for s in outs] if n_out > 1
        else full(outs[0].shape),
        out_shape=out_shape,
    )


def _proj1(x, Wl, Wr, ear):
    def body(x_ref, wl_ref, wr_ref, ear_ref, xlr_ref, mean_ref):
        xv = x_ref[...]
        xl = jnp.dot(xv, wl_ref[...], preferred_element_type=jnp.float32)
        xr = jnp.dot(xv, wr_ref[...], preferred_element_type=jnp.float32)
        xlr_ref[...] = jnp.concatenate([xl, xr], axis=1)
        mean_ref[...] = jnp.sum(ear_ref[...], axis=1,
                                keepdims=True) * (1.0 / _E)

    return _tc_full(
        body,
        [x.shape, Wl.shape, Wr.shape, ear.shape],
        (jax.ShapeDtypeStruct((_N, 128), jnp.float32),
         jax.ShapeDtypeStruct((1, 1), jnp.float32)),
        n_out=2,
    )(x, Wl, Wr, ear)


def _node_proj(acc, Snum, Sden, b, g, be, Wl, Wr, pad):
    C2 = Wl.shape[1]

    def body(acc_ref, sn_ref, sd_ref, b_ref, g_ref, be_ref, wl_ref, wr_ref,
             xlr_ref):
        accs = acc_ref[0] + acc_ref[1]
        num = jnp.dot(accs, sn_ref[...], preferred_element_type=jnp.float32)
        den = jnp.dot(accs, sd_ref[...], preferred_element_type=jnp.float32)
        o = num / den + b_ref[...]
        mu = jnp.mean(o, axis=-1, keepdims=True)
        var = jnp.mean((o - mu) * (o - mu), axis=-1, keepdims=True)
        o = (o - mu) * lax.rsqrt(var + _EPS) * g_ref[...] + be_ref[...]
        o = jnp.where(o > 0, o, jnp.exp(o) - 1.0)
        xl = jnp.dot(o, wl_ref[...], preferred_element_type=jnp.float32)
        xr = jnp.dot(o, wr_ref[...], preferred_element_type=jnp.float32)
        if pad:
            z = jnp.zeros_like(xl)
            xlr_ref[...] = jnp.concatenate([xl, z, xr, z], axis=1)
        else:
            xlr_ref[...] = jnp.concatenate([xl, xr], axis=1)

    return _tc_full(
        body,
        [acc.shape, Snum.shape, Sden.shape, b.shape, g.shape, be.shape,
         Wl.shape, Wr.shape],
        jax.ShapeDtypeStruct((_N, 128), jnp.float32),
    )(acc, Snum, Sden, b, g, be, Wl, Wr)


def _node_final(acc, Snum, Sden, b, g, be):
    def body(acc_ref, sn_ref, sd_ref, b_ref, g_ref, be_ref, emb_ref):
        accs = acc_ref[0] + acc_ref[1]
        num = jnp.dot(accs, sn_ref[...], preferred_element_type=jnp.float32)
        den = jnp.dot(accs, sd_ref[...], preferred_element_type=jnp.float32)
        o = num / den + b_ref[...]
        mu = jnp.mean(o, axis=-1, keepdims=True)
        var = jnp.mean((o - mu) * (o - mu), axis=-1, keepdims=True)
        emb_ref[...] = (o - mu) * lax.rsqrt(var + _EPS) * g_ref[...] \
            + be_ref[...]

    return _tc_full(
        body,
        [acc.shape, Snum.shape, Sden.shape, b.shape, g.shape, be.shape],
        jax.ShapeDtypeStruct((_N, 32), jnp.float32),
    )(acc, Snum, Sden, b, g, be)


def _decode(embT, D1aT, D1bT, d1b_c, ga_c, ba_c, D2T, d2b_c, gb_c, bb_c,
            D3_c, d3b_c):
    def body(embT_ref, d1a_ref, d1b_ref, d1bias_ref, ga_ref, ba_ref, d2_ref,
             d2b_ref, gb_ref, bb_ref, d3_ref, d3b_ref, out_ref, A_s, B_s):
        k = pl.program_id(0)

        @pl.when(k == 0)
        def _():
            et = embT_ref[...]
            A_s[...] = jnp.dot(d1a_ref[...], et,
                               preferred_element_type=jnp.float32) \
                + d1bias_ref[...]
            B_s[...] = jnp.dot(d1b_ref[...], et,
                               preferred_element_type=jnp.float32)

        B = B_s[...]
        Afull = A_s[...]
        ga_v = ga_ref[...]
        ba_v = ba_ref[...]
        gb_v = gb_ref[...]
        bb_v = bb_ref[...]
        d2b_v = d2b_ref[...]
        d3_v = d3_ref[...]
        iota_cols = lax.broadcasted_iota(jnp.int32, (64, _N), 1)
        for r in range(_BI):
            sel = iota_cols == (k * _BI + r)
            a = jnp.sum(jnp.where(sel, Afull, 0.0), axis=1, keepdims=True)
            h = B + a
            mu = jnp.mean(h, axis=0, keepdims=True)
            var = jnp.mean((h - mu) * (h - mu), axis=0, keepdims=True)
            h = (h - mu) * lax.rsqrt(var + _EPS) * ga_v + ba_v
            h = jnp.maximum(h, 0.1 * h)
            h2 = jnp.dot(d2_ref[...], h,
                         preferred_element_type=jnp.float32) + d2b_v
            mu2 = jnp.mean(h2, axis=0, keepdims=True)
            var2 = jnp.mean((h2 - mu2) * (h2 - mu2), axis=0, keepdims=True)
            h2 = (h2 - mu2) * lax.rsqrt(var2 + _EPS) * gb_v + bb_v
            h2 = jnp.maximum(h2, 0.1 * h2)
            logit = jnp.sum(h2 * d3_v, axis=0, keepdims=True) + d3b_ref[...]
            out_ref[r:r + 1, :] = 1.0 / (1.0 + jnp.exp(-logit))

    full = lambda shape: pl.BlockSpec(shape, lambda k: tuple(0 for _ in shape))
    return pl.pallas_call(
        body,
        grid=(_NBI,),
        in_specs=[
            full((32, _N)),
            full((64, 32)),
            full((64, 32)),
            full((64, 1)),
            full((64, 1)),
            full((64, 1)),
            full((32, 64)),
            full((32, 1)),
            full((32, 1)),
            full((32, 1)),
            full((32, 1)),
            full((1, 1)),
        ],
        out_specs=pl.BlockSpec((_BI, _N), lambda k: (k, 0)),
        out_shape=jax.ShapeDtypeStruct((_N, _N), jnp.float32),
        scratch_shapes=[
            pltpu.VMEM((64, _N), jnp.float32),
            pltpu.VMEM((64, _N), jnp.float32),
        ],
        compiler_params=pltpu.CompilerParams(
            dimension_semantics=("arbitrary",)),
    )(embT, D1aT, D1bT, d1b_c, ga_c, ba_c, D2T, d2b_c, gb_c, bb_c,
      D3_c, d3b_c)


def _bcast16(v):
    return jnp.broadcast_to(v.reshape(-1, 1), (v.size, 16)) \
        .astype(jnp.float32)


def _sel_mats(heads, C):
    ch = C // heads
    cols = jnp.arange(C)
    rows = jnp.arange(128)
    Snum = (rows[:, None] == cols[None, :]).astype(jnp.float32)
    Sden = ((rows[:, None] - 64) == (cols[None, :] // ch)).astype(jnp.float32)
    return Snum, Sden


@jax.jit
def kernel(x, edge_index, edge_attr, W1l, W1r, a1, We1, b1, W2l, W2r, a2, We2,
           b2, W3l, W3r, a3, We3, b3, g1, be1, g2, be2, g3, be3, D1, d1b, ga,
           ba, D2, d2b, gb, bb, D3, d3b):
    src, dst = edge_index[0], edge_index[1]
    loop = jnp.arange(_N, dtype=src.dtype)
    s3 = jnp.concatenate([src, loop]).reshape(_NW, _NCH, 16)
    d3 = jnp.concatenate([dst, loop]).reshape(_NW, _NCH, 16)
    ear = edge_attr.reshape(1, _E)

    row = lambda v: v.reshape(1, -1)
    col = lambda v: v.reshape(-1, 1)

    xlr1, mean_ea = _proj1(x, W1l, W1r, ear)
    ea3 = jnp.concatenate(
        [edge_attr[:, 0], jnp.broadcast_to(mean_ea.reshape(1), (_N,))]
    ).reshape(_NW, _NCH, 16)

    Snum4, Sden4 = _sel_mats(4, 64)
    Snum1, Sden1 = _sel_mats(1, 32)

    acc1 = _sc_edge_pass(xlr1, s3, d3, ea3, _bcast16(a1), _bcast16(We1),
                         4, 64)
    xlr2 = _node_proj(acc1, Snum4, Sden4, row(b1), row(g1), row(be1),
                      W2l, W2r, False)
    acc2 = _sc_edge_pass(xlr2, s3, d3, ea3, _bcast16(a2), _bcast16(We2),
                         4, 64)
    xlr3 = _node_proj(acc2, Snum4, Sden4, row(b2), row(g2), row(be2),
                      W3l, W3r, True)
    acc3 = _sc_edge_pass(xlr3, s3, d3, ea3, _bcast16(a3), _bcast16(We3),
                         1, 32)
    emb = _node_final(acc3, Snum1, Sden1, row(b3), row(g3), row(be3))

    weights = _decode(emb.T, D1[:32].T, D1[32:].T, col(d1b), col(ga),
                      col(ba), D2.T, col(d2b), col(gb), col(bb),
                      D3.reshape(32, 1), d3b.reshape(1, 1))
    return (weights, emb)


# speedup vs baseline: 1.6577x; 1.0518x over previous
import functools

import jax
import jax.numpy as jnp
from jax import lax
from jax.experimental import pallas as pl
from jax.experimental.pallas import tpu as pltpu
from jax.experimental.pallas import tpu_sc as plsc

_N = 1024
_E = 16384
_ET = _E + _N
_NW = 32
_NCH = (_ET // _NW) // 16
_BI = 8
_NBI = _N // _BI
_EPS = 1e-5


def _sc_edge_pass(xlr, s3, d3, ea3, attB, WeB, heads, C):
    H = heads
    ch = C // H
    mesh = plsc.VectorSubcoreMesh(core_axis_name="c", subcore_axis_name="s")

    @functools.partial(
        pl.kernel,
        mesh=mesh,
        out_type=jax.ShapeDtypeStruct((2, _N, 128), jnp.float32),
        scratch_types=[
            pltpu.VMEM((_NCH, 16), jnp.int32),
            pltpu.VMEM((_NCH, 16), jnp.int32),
            pltpu.VMEM((_NCH, 16), jnp.float32),
            pltpu.VMEM((C, 16), jnp.float32),
            pltpu.VMEM((C, 16), jnp.float32),
            pltpu.VMEM((16, 128), jnp.float32),
            pltpu.VMEM((16, 128), jnp.float32),
            pltpu.VMEM((16, 128), jnp.float32),
            pltpu.VMEM((16, 128), jnp.float32),
            pltpu.VMEM((16, 128), jnp.float32),
            pltpu.VMEM((16, 128), jnp.float32),
            pltpu.VMEM_SHARED((_N, 128), jnp.float32),
            pltpu.VMEM_SHARED((_N, 128), jnp.float32),
            pltpu.SemaphoreType.DMA,
            pltpu.SemaphoreType.DMA,
            pltpu.SemaphoreType.DMA,
            pltpu.SemaphoreType.DMA,
        ],
        compiler_params=pltpu.CompilerParams(needs_layout_passes=False),
    )
    def k(xlr_hbm, s_hbm, d_hbm, ea_hbm, attB_hbm, WeB_hbm, acc_hbm,
          s_v, d_v, ea_v, attB_v, WeB_v, xs_b0, xd_b0, xs_b1, xd_b1,
          n_b0, n_b1, acc_sh, tab_sh, gs0, gd0, gs1, gd1):
        cid = lax.axis_index("c")
        sid = lax.axis_index("s")
        wid = sid * 2 + cid

        pltpu.sync_copy(s_hbm.at[wid], s_v)
        pltpu.sync_copy(d_hbm.at[wid], d_v)
        pltpu.sync_copy(ea_hbm.at[wid], ea_v)
        pltpu.sync_copy(attB_hbm, attB_v)
        pltpu.sync_copy(WeB_hbm, WeB_v)
        my_rows = pl.ds(sid * 64, 64)
        pltpu.sync_copy(xlr_hbm.at[my_rows], tab_sh.at[my_rows])

        zero = jnp.zeros((16,), jnp.float32)
        for r in range(16):
            for cc in range(8):
                n_b0[r, pl.ds(cc * 16, 16)] = zero
                n_b1[r, pl.ds(cc * 16, 16)] = zero
        for q in range(4):
            pltpu.sync_copy(n_b0, acc_sh.at[pl.ds(sid * 64 + q * 16, 16)])
        plsc.subcore_barrier()

        lane = lax.iota(jnp.int32, 16)

        def issue(j, xsb, xdb, ssem, dsem):
            pltpu.async_copy(tab_sh.at[s_v[j]], xsb, ssem)
            pltpu.async_copy(tab_sh.at[d_v[j]], xdb, dsem)

        def wait(xsb, xdb, ssem, dsem):
            pltpu.make_async_copy(tab_sh.at[pl.ds(0, 16)], xsb, ssem).wait()
            pltpu.make_async_copy(tab_sh.at[pl.ds(0, 16)], xdb, dsem).wait()

        def compute(j, xsb, xdb, nb):
            eav = ea_v[j]
            logits = [jnp.zeros((16,), jnp.float32) for _ in range(H)]
            for c in range(C):
                cl = jnp.full((16,), c, jnp.int32)
                cr = jnp.full((16,), 64 + c, jnp.int32)
                xlc = plsc.load_gather(xsb, [lane, cl])
                xrc = plsc.load_gather(xdb, [lane, cr])
                mc = xlc + xrc + eav * WeB_v[c]
                mc = jnp.maximum(mc, 0.2 * mc)
                logits[c // ch] = logits[c // ch] + mc * attB_v[c]
            exl = [jnp.exp(lg) for lg in logits]
            for c in range(C):
                cl = jnp.full((16,), c, jnp.int32)
                xlc = plsc.load_gather(xsb, [lane, cl])
                plsc.store_scatter(nb, [lane, cl], xlc * exl[c // ch])
            for h in range(H):
                ch_ = jnp.full((16,), 64 + h, jnp.int32)
                plsc.store_scatter(nb, [lane, ch_], exl[h])
            pltpu.sync_copy(nb, acc_sh.at[d_v.at[j]], add=True)

        issue(0, xs_b0, xd_b0, gs0, gd0)

        def pair(i, carry):
            j0 = 2 * i
            j1 = j0 + 1
            j2 = jnp.minimum(j0 + 2, _NCH - 1)
            issue(j1, xs_b1, xd_b1, gs1, gd1)
            wait(xs_b0, xd_b0, gs0, gd0)
            compute(j0, xs_b0, xd_b0, n_b0)
            issue(j2, xs_b0, xd_b0, gs0, gd0)
            wait(xs_b1, xd_b1, gs1, gd1)
            compute(j1, xs_b1, xd_b1, n_b1)
            return carry

        lax.fori_loop(0, _NCH // 2, pair, 0)
        wait(xs_b0, xd_b0, gs0, gd0)
        plsc.subcore_barrier()
        rows = pl.ds(sid * 64, 64)
        pltpu.sync_copy(acc_sh.at[rows], acc_hbm.at[cid, rows])

    return k(xlr, s3, d3, ea3, attB, WeB)


def _tc_full(body, in_shapes, out_shape, n_out=1):
    full = lambda shape: pl.BlockSpec(shape, lambda: tuple(0 for _ in shape))
    outs = out_shape if isinstance(out_shape, tuple) else (out_shape,)
    return pl.pallas_call(
        body,
        in_specs=[full(s) for s in in_shapes],
        out_specs=[full(s.shape) for s in outs] if n_out > 1
        else full(outs[0].shape),
        out_shape=out_shape,
    )


def _proj1(x, Wl, Wr, ear):
    def body(x_ref, wl_ref, wr_ref, ear_ref, xlr_ref, mean_ref):
        xv = x_ref[...]
        xl = jnp.dot(xv, wl_ref[...], preferred_element_type=jnp.float32)
        xr = jnp.dot(xv, wr_ref[...], preferred_element_type=jnp.float32)
        xlr_ref[...] = jnp.concatenate([xl, xr], axis=1)
        mean_ref[...] = jnp.sum(ear_ref[...], axis=1,
                                keepdims=True) * (1.0 / _E)

    return _tc_full(
        body,
        [x.shape, Wl.shape, Wr.shape, ear.shape],
        (jax.ShapeDtypeStruct((_N, 128), jnp.float32),
         jax.ShapeDtypeStruct((1, 1), jnp.float32)),
        n_out=2,
    )(x, Wl, Wr, ear)


def _node_proj(acc, Snum, Sden, b, g, be, Wl, Wr, pad):
    C2 = Wl.shape[1]

    def body(acc_ref, sn_ref, sd_ref, b_ref, g_ref, be_ref, wl_ref, wr_ref,
             xlr_ref):
        accs = acc_ref[0] + acc_ref[1]
        num = jnp.dot(accs, sn_ref[...], preferred_element_type=jnp.float32)
        den = jnp.dot(accs, sd_ref[...], preferred_element_type=jnp.float32)
        o = num / den + b_ref[...]
        mu = jnp.mean(o, axis=-1, keepdims=True)
        var = jnp.mean((o - mu) * (o - mu), axis=-1, keepdims=True)
        o = (o - mu) * lax.rsqrt(var + _EPS) * g_ref[...] + be_ref[...]
        o = jnp.where(o > 0, o, jnp.exp(o) - 1.0)
        xl = jnp.dot(o, wl_ref[...], preferred_element_type=jnp.float32)
        xr = jnp.dot(o, wr_ref[...], preferred_element_type=jnp.float32)
        if pad:
            z = jnp.zeros_like(xl)
            xlr_ref[...] = jnp.concatenate([xl, z, xr, z], axis=1)
        else:
            xlr_ref[...] = jnp.concatenate([xl, xr], axis=1)

    return _tc_full(
        body,
        [acc.shape, Snum.shape, Sden.shape, b.shape, g.shape, be.shape,
         Wl.shape, Wr.shape],
        jax.ShapeDtypeStruct((_N, 128), jnp.float32),
    )(acc, Snum, Sden, b, g, be, Wl, Wr)


def _node_final(acc, Snum, Sden, b, g, be):
    def body(acc_ref, sn_ref, sd_ref, b_ref, g_ref, be_ref, emb_ref):
        accs = acc_ref[0] + acc_ref[1]
        num = jnp.dot(accs, sn_ref[...], preferred_element_type=jnp.float32)
        den = jnp.dot(accs, sd_ref[...], preferred_element_type=jnp.float32)
        o = num / den + b_ref[...]
        mu = jnp.mean(o, axis=-1, keepdims=True)
        var = jnp.mean((o - mu) * (o - mu), axis=-1, keepdims=True)
        emb_ref[...] = (o - mu) * lax.rsqrt(var + _EPS) * g_ref[...] \
            + be_ref[...]

    return _tc_full(
        body,
        [acc.shape, Snum.shape, Sden.shape, b.shape, g.shape, be.shape],
        jax.ShapeDtypeStruct((_N, 32), jnp.float32),
    )(acc, Snum, Sden, b, g, be)


def _decode(embT, D1aT, D1bT, d1b_c, ga_c, ba_c, D2T, d2b_c, gb_c, bb_c,
            D3_c, d3b_c):
    def body(embT_ref, d1a_ref, d1b_ref, d1bias_ref, ga_ref, ba_ref, d2_ref,
             d2b_ref, gb_ref, bb_ref, d3_ref, d3b_ref, out_ref, A_s, B_s):
        k = pl.program_id(0)

        @pl.when(k == 0)
        def _():
            et = embT_ref[...]
            A_s[...] = jnp.dot(d1a_ref[...], et,
                               preferred_element_type=jnp.float32) \
                + d1bias_ref[...]
            B_s[...] = jnp.dot(d1b_ref[...], et,
                               preferred_element_type=jnp.float32)

        B = B_s[...]
        Afull = A_s[...]
        ga_v = ga_ref[...]
        ba_v = ba_ref[...]
        gb_v = gb_ref[...]
        bb_v = bb_ref[...]
        d2b_v = d2b_ref[...]
        d3_v = d3_ref[...]
        iota_cols = lax.broadcasted_iota(jnp.int32, (64, _N), 1)
        for r in range(_BI):
            sel = iota_cols == (k * _BI + r)
            a = jnp.sum(jnp.where(sel, Afull, 0.0), axis=1, keepdims=True)
            h = B + a
            mu = jnp.mean(h, axis=0, keepdims=True)
            var = jnp.mean((h - mu) * (h - mu), axis=0, keepdims=True)
            h = (h - mu) * lax.rsqrt(var + _EPS) * ga_v + ba_v
            h = jnp.maximum(h, 0.1 * h)
            h2 = jnp.dot(d2_ref[...], h,
                         preferred_element_type=jnp.float32) + d2b_v
            mu2 = jnp.mean(h2, axis=0, keepdims=True)
            var2 = jnp.mean((h2 - mu2) * (h2 - mu2), axis=0, keepdims=True)
            h2 = (h2 - mu2) * lax.rsqrt(var2 + _EPS) * gb_v + bb_v
            h2 = jnp.maximum(h2, 0.1 * h2)
            logit = jnp.sum(h2 * d3_v, axis=0, keepdims=True) + d3b_ref[...]
            out_ref[r:r + 1, :] = 1.0 / (1.0 + jnp.exp(-logit))

    full = lambda shape: pl.BlockSpec(shape, lambda k: tuple(0 for _ in shape))
    return pl.pallas_call(
        body,
        grid=(_NBI,),
        in_specs=[
            full((32, _N)),
            full((64, 32)),
            full((64, 32)),
            full((64, 1)),
            full((64, 1)),
            full((64, 1)),
            full((32, 64)),
            full((32, 1)),
            full((32, 1)),
            full((32, 1)),
            full((32, 1)),
            full((1, 1)),
        ],
        out_specs=pl.BlockSpec((_BI, _N), lambda k: (k, 0)),
        out_shape=jax.ShapeDtypeStruct((_N, _N), jnp.float32),
        scratch_shapes=[
            pltpu.VMEM((64, _N), jnp.float32),
            pltpu.VMEM((64, _N), jnp.float32),
        ],
        compiler_params=pltpu.CompilerParams(
            dimension_semantics=("arbitrary",)),
    )(embT, D1aT, D1bT, d1b_c, ga_c, ba_c, D2T, d2b_c, gb_c, bb_c,
      D3_c, d3b_c)


def _bcast16(v):
    return jnp.broadcast_to(v.reshape(-1, 1), (v.size, 16)) \
        .astype(jnp.float32)


def _sel_mats(heads, C):
    ch = C // heads
    cols = jnp.arange(C)
    rows = jnp.arange(128)
    Snum = (rows[:, None] == cols[None, :]).astype(jnp.float32)
    Sden = ((rows[:, None] - 64) == (cols[None, :] // ch)).astype(jnp.float32)
    return Snum, Sden


@jax.jit
def kernel(x, edge_index, edge_attr, W1l, W1r, a1, We1, b1, W2l, W2r, a2, We2,
           b2, W3l, W3r, a3, We3, b3, g1, be1, g2, be2, g3, be3, D1, d1b, ga,
           ba, D2, d2b, gb, bb, D3, d3b):
    src, dst = edge_index[0], edge_index[1]
    loop = jnp.arange(_N, dtype=src.dtype)
    s3 = jnp.concatenate([src, loop]).reshape(_NW, _NCH, 16)
    d3 = jnp.concatenate([dst, loop]).reshape(_NW, _NCH, 16)
    ear = edge_attr.reshape(1, _E)

    row = lambda v: v.reshape(1, -1)
    col = lambda v: v.reshape(-1, 1)

    xlr1, mean_ea = _proj1(x, W1l, W1r, ear)
    ea3 = jnp.concatenate(
        [edge_attr[:, 0], jnp.broadcast_to(mean_ea.reshape(1), (_N,))]
    ).reshape(_NW, _NCH, 16)

    Snum4, Sden4 = _sel_mats(4, 64)
    Snum1, Sden1 = _sel_mats(1, 32)

    acc1 = _sc_edge_pass(xlr1, s3, d3, ea3, _bcast16(a1), _bcast16(We1),
                         4, 64)
    xlr2 = _node_proj(acc1, Snum4, Sden4, row(b1), row(g1), row(be1),
                      W2l, W2r, False)
    acc2 = _sc_edge_pass(xlr2, s3, d3, ea3, _bcast16(a2), _bcast16(We2),
                         4, 64)
    xlr3 = _node_proj(acc2, Snum4, Sden4, row(b2), row(g2), row(be2),
                      W3l, W3r, True)
    acc3 = _sc_edge_pass(xlr3, s3, d3, ea3, _bcast16(a3), _bcast16(We3),
                         1, 32)
    emb = _node_final(acc3, Snum1, Sden1, row(b3), row(g3), row(be3))

    weights = _decode(emb.T, D1[:32].T, D1[32:].T, col(d1b), col(ga),
                      col(ba), D2.T, col(d2b), col(gb), col(bb),
                      D3.reshape(32, 1), d3b.reshape(1, 1))
    return (weights, emb)
